# Initial kernel scaffold; baseline (speedup 1.0000x reference)
import jax
import jax.numpy as jnp
from jax.experimental import pallas as pl


def kernel(x, edge_index):
    raise NotImplementedError("write your pallas kernel here")


# speedup vs baseline: 3.7555x; 3.7555x over previous
import functools

import jax
import jax.numpy as jnp
from jax import lax
from jax.experimental import pallas as pl
from jax.experimental.pallas import tpu as pltpu
from jax.experimental.pallas import tpu_sc as plsc

N_NODES = 10000
D_FEAT = 256
N_EDGES = 160000

NC = 2
NS = 16
DH = D_FEAT // NC
E_PER_TILE = N_EDGES // NS
B = 80
NCHUNK = E_PER_TILE // B
ROWS_PER_TILE = 624
ROWS_TAIL = N_NODES - NS * ROWS_PER_TILE


def _mp_body(x2, src_off, dst_all, zeros, out, acc, idx_v, dst_v, rows_v):
    c = lax.axis_index("c")
    s = lax.axis_index("s")

    row0 = s * ROWS_PER_TILE
    pltpu.sync_copy(zeros.at[pl.ds(row0, ROWS_PER_TILE)],
                    acc.at[pl.ds(row0, ROWS_PER_TILE)])

    @pl.when(s == 0)
    def _zero_tail():
        pltpu.sync_copy(zeros.at[pl.ds(NS * ROWS_PER_TILE, ROWS_TAIL)],
                        acc.at[pl.ds(NS * ROWS_PER_TILE, ROWS_TAIL)])

    plsc.subcore_barrier()

    base_e = c * N_EDGES + s * E_PER_TILE

    def chunk(j, carry):
        start = base_e + j * B
        pltpu.sync_copy(src_off.at[pl.ds(start, B)], idx_v)
        pltpu.sync_copy(dst_all.at[pl.ds(start - c * N_EDGES, B)], dst_v)
        pltpu.sync_copy(x2.at[idx_v], rows_v)
        pltpu.sync_copy(rows_v, acc.at[dst_v], add=True)
        return carry

    lax.fori_loop(0, NCHUNK, chunk, 0)

    plsc.subcore_barrier()
    pltpu.sync_copy(acc.at[pl.ds(row0, ROWS_PER_TILE)],
                    out.at[pl.ds(row0, ROWS_PER_TILE), pl.ds(c * DH, DH)])

    @pl.when(s == 0)
    def _out_tail():
        pltpu.sync_copy(
            acc.at[pl.ds(NS * ROWS_PER_TILE, ROWS_TAIL)],
            out.at[pl.ds(NS * ROWS_PER_TILE, ROWS_TAIL), pl.ds(c * DH, DH)])


_mp_call = functools.partial(
    pl.kernel,
    out_type=jax.ShapeDtypeStruct((N_NODES, D_FEAT), jnp.float32),
    mesh=plsc.VectorSubcoreMesh(core_axis_name="c", subcore_axis_name="s",
                                num_cores=NC, num_subcores=NS),
    scratch_types=[
        pltpu.VMEM_SHARED((N_NODES, DH), jnp.float32),
        pltpu.VMEM((B,), jnp.int32),
        pltpu.VMEM((B,), jnp.int32),
        pltpu.VMEM((B, DH), jnp.float32),
    ],
)(_mp_body)


def kernel(x, edge_index):
    ei = edge_index.astype(jnp.int32)
    dst = ei[0]
    src = ei[1]
    x2 = jnp.concatenate([x[:, :DH], x[:, DH:]], axis=0)
    src_off = jnp.concatenate([src, src + N_NODES])
    zeros = jnp.zeros((N_NODES, DH), jnp.float32)
    return _mp_call(x2, src_off, dst, zeros)


# speedup vs baseline: 8.3524x; 2.2241x over previous
import functools

import jax
import jax.numpy as jnp
from jax import lax
from jax.experimental import pallas as pl
from jax.experimental.pallas import tpu as pltpu
from jax.experimental.pallas import tpu_sc as plsc

N_NODES = 10000
D_FEAT = 256
N_EDGES = 160000

NC = 2
NS = 16
DH = D_FEAT // NC
E_PER_TILE = N_EDGES // NS
B = 80
NCHUNK = E_PER_TILE // B
NBUF = 4
LA = 2
NBUFI = 8
LAI = 6
ROWS_PER_TILE = 624
ROWS_TAIL = N_NODES - NS * ROWS_PER_TILE


def _mp_body(x2, src_off, dst_all, zeros, out,
             acc, sidx, didx, rows, gsem, ssem, isem_s, isem_d):
    c = lax.axis_index("c")
    s = lax.axis_index("s")

    src_rows = src_off.at[c * NS + s]
    dst_rows = dst_all.at[s]

    def start_idx(j):
        bi = lax.rem(j, NBUFI)
        pltpu.async_copy(src_rows.at[j], sidx.at[bi], isem_s.at[bi])
        pltpu.async_copy(dst_rows.at[j], didx.at[bi], isem_d.at[bi])

    def wait_idx(j):
        bi = lax.rem(j, NBUFI)
        pltpu.make_async_copy(src_rows.at[j], sidx.at[bi],
                              isem_s.at[bi]).wait()
        pltpu.make_async_copy(dst_rows.at[j], didx.at[bi],
                              isem_d.at[bi]).wait()

    def start_gather(j, b):
        bi = lax.rem(j, NBUFI)
        pltpu.async_copy(x2.at[sidx.at[bi]], rows.at[b], gsem.at[b])

    def wait_gather(j, b):
        bi = lax.rem(j, NBUFI)
        pltpu.make_async_copy(x2.at[sidx.at[bi]], rows.at[b],
                              gsem.at[b]).wait()

    def start_scatter(j, b):
        bi = lax.rem(j, NBUFI)
        pltpu.async_copy(rows.at[b], acc.at[didx.at[bi]], ssem.at[b],
                         add=True)

    def wait_scatter(j, b):
        bi = lax.rem(j, NBUFI)
        pltpu.make_async_copy(rows.at[b], acc.at[didx.at[bi]],
                              ssem.at[b]).wait()

    for k in range(LAI):
        start_idx(k)

    row0 = s * ROWS_PER_TILE
    pltpu.sync_copy(zeros.at[pl.ds(row0, ROWS_PER_TILE)],
                    acc.at[pl.ds(row0, ROWS_PER_TILE)])

    @pl.when(s == 0)
    def _zero_tail():
        pltpu.sync_copy(zeros.at[pl.ds(NS * ROWS_PER_TILE, ROWS_TAIL)],
                        acc.at[pl.ds(NS * ROWS_PER_TILE, ROWS_TAIL)])

    plsc.subcore_barrier()

    for k in range(LA):
        wait_idx(k)
        start_gather(k, k)

    def chunk(j, carry):
        b = lax.rem(j, NBUF)
        wait_gather(j, b)
        start_scatter(j, b)

        @pl.when(j >= LA)
        def _drain():
            wait_scatter(j - LA, lax.rem(j - LA, NBUF))

        @pl.when(j + LAI < NCHUNK)
        def _pf_idx():
            start_idx(j + LAI)

        @pl.when(j + LA < NCHUNK)
        def _pf_gather():
            wait_idx(j + LA)
            start_gather(j + LA, lax.rem(j + LA, NBUF))

        return carry

    lax.fori_loop(0, NCHUNK, chunk, 0)

    for j in range(NCHUNK - LA, NCHUNK):
        wait_scatter(j, j % NBUF)

    plsc.subcore_barrier()
    pltpu.sync_copy(acc.at[pl.ds(row0, ROWS_PER_TILE)],
                    out.at[pl.ds(row0, ROWS_PER_TILE), pl.ds(c * DH, DH)])

    @pl.when(s == 0)
    def _out_tail():
        pltpu.sync_copy(
            acc.at[pl.ds(NS * ROWS_PER_TILE, ROWS_TAIL)],
            out.at[pl.ds(NS * ROWS_PER_TILE, ROWS_TAIL), pl.ds(c * DH, DH)])


_mp_call = functools.partial(
    pl.kernel,
    out_type=jax.ShapeDtypeStruct((N_NODES, D_FEAT), jnp.float32),
    mesh=plsc.VectorSubcoreMesh(core_axis_name="c", subcore_axis_name="s",
                                num_cores=NC, num_subcores=NS),
    scratch_types=[
        pltpu.VMEM_SHARED((N_NODES, DH), jnp.float32),
        pltpu.VMEM((NBUFI, B), jnp.int32),
        pltpu.VMEM((NBUFI, B), jnp.int32),
        pltpu.VMEM((NBUF, B, DH), jnp.float32),
        pltpu.SemaphoreType.DMA((NBUF,)),
        pltpu.SemaphoreType.DMA((NBUF,)),
        pltpu.SemaphoreType.DMA((NBUFI,)),
        pltpu.SemaphoreType.DMA((NBUFI,)),
    ],
)(_mp_body)


def kernel(x, edge_index):
    ei = edge_index.astype(jnp.int32)
    dst = ei[0]
    src = ei[1]
    x2 = jnp.concatenate([x[:, :DH], x[:, DH:]], axis=0)
    src_off = jnp.concatenate([src, src + N_NODES]).reshape(
        NC * NS, NCHUNK, B)
    dst_r = dst.reshape(NS, NCHUNK, B)
    zeros = jnp.zeros((N_NODES, DH), jnp.float32)
    return _mp_call(x2, src_off, dst_r, zeros)


# speedup vs baseline: 9.5104x; 1.1386x over previous
import functools

import jax
import jax.numpy as jnp
from jax import lax
from jax.experimental import pallas as pl
from jax.experimental.pallas import tpu as pltpu
from jax.experimental.pallas import tpu_sc as plsc

N_NODES = 10000
D_FEAT = 256
N_EDGES = 160000

NC = 2
NS = 16
DH = D_FEAT // NC
E_PER_TILE = N_EDGES // NS
B = 80
NCHUNK = E_PER_TILE // B
NBUF = 4
LA = 2
NBUFI = 8
LAI = 6
ROWS_PER_TILE = 624
ROWS_TAIL = N_NODES - NS * ROWS_PER_TILE


def _mp_body(x, src_all, dst_all, zeros, out,
             acc, sidx, didx, rows, gsem, ssem, isem_s, isem_d):
    c = lax.axis_index("c")
    s = lax.axis_index("s")
    col0 = c * DH

    base_e = s * E_PER_TILE

    def start_idx(j):
        bi = lax.rem(j, NBUFI)
        st = base_e + j * B
        pltpu.async_copy(src_all.at[pl.ds(st, B)], sidx.at[bi], isem_s.at[bi])
        pltpu.async_copy(dst_all.at[pl.ds(st, B)], didx.at[bi], isem_d.at[bi])

    def wait_idx(j):
        bi = lax.rem(j, NBUFI)
        st = base_e + j * B
        pltpu.make_async_copy(src_all.at[pl.ds(st, B)], sidx.at[bi],
                              isem_s.at[bi]).wait()
        pltpu.make_async_copy(dst_all.at[pl.ds(st, B)], didx.at[bi],
                              isem_d.at[bi]).wait()

    def start_gather(j, b):
        bi = lax.rem(j, NBUFI)
        pltpu.async_copy(x.at[sidx.at[bi], pl.ds(col0, DH)], rows.at[b],
                         gsem.at[b])

    def wait_gather(j, b):
        bi = lax.rem(j, NBUFI)
        pltpu.make_async_copy(x.at[sidx.at[bi], pl.ds(col0, DH)], rows.at[b],
                              gsem.at[b]).wait()

    def start_scatter(j, b):
        bi = lax.rem(j, NBUFI)
        pltpu.async_copy(rows.at[b], acc.at[didx.at[bi]], ssem.at[b],
                         add=True)

    def wait_scatter(j, b):
        bi = lax.rem(j, NBUFI)
        pltpu.make_async_copy(rows.at[b], acc.at[didx.at[bi]],
                              ssem.at[b]).wait()

    for k in range(LAI):
        start_idx(k)

    row0 = s * ROWS_PER_TILE
    for r in range(ROWS_PER_TILE // ROWS_Z):
        pltpu.sync_copy(zeros, acc.at[pl.ds(row0 + r * ROWS_Z, ROWS_Z)])

    @pl.when(s == 0)
    def _zero_tail():
        pltpu.sync_copy(zeros.at[pl.ds(0, ROWS_TAIL)],
                        acc.at[pl.ds(NS * ROWS_PER_TILE, ROWS_TAIL)])

    plsc.subcore_barrier()

    for k in range(LA):
        wait_idx(k)
        start_gather(k, k)

    def chunk(j, carry):
        b = lax.rem(j, NBUF)
        wait_gather(j, b)
        start_scatter(j, b)

        @pl.when(j >= LA)
        def _drain():
            wait_scatter(j - LA, lax.rem(j - LA, NBUF))

        @pl.when(j + LAI < NCHUNK)
        def _pf_idx():
            start_idx(j + LAI)

        @pl.when(j + LA < NCHUNK)
        def _pf_gather():
            wait_idx(j + LA)
            start_gather(j + LA, lax.rem(j + LA, NBUF))

        return carry

    lax.fori_loop(0, NCHUNK, chunk, 0)

    for j in range(NCHUNK - LA, NCHUNK):
        wait_scatter(j, j % NBUF)

    plsc.subcore_barrier()
    pltpu.sync_copy(acc.at[pl.ds(row0, ROWS_PER_TILE)],
                    out.at[pl.ds(row0, ROWS_PER_TILE), pl.ds(col0, DH)])

    @pl.when(s == 0)
    def _out_tail():
        pltpu.sync_copy(
            acc.at[pl.ds(NS * ROWS_PER_TILE, ROWS_TAIL)],
            out.at[pl.ds(NS * ROWS_PER_TILE, ROWS_TAIL), pl.ds(col0, DH)])


ROWS_Z = 208

_mp_call = functools.partial(
    pl.kernel,
    out_type=jax.ShapeDtypeStruct((N_NODES, D_FEAT), jnp.float32),
    mesh=plsc.VectorSubcoreMesh(core_axis_name="c", subcore_axis_name="s",
                                num_cores=NC, num_subcores=NS),
    scratch_types=[
        pltpu.VMEM_SHARED((N_NODES, DH), jnp.float32),
        pltpu.VMEM((NBUFI, B), jnp.int32),
        pltpu.VMEM((NBUFI, B), jnp.int32),
        pltpu.VMEM((NBUF, B, DH), jnp.float32),
        pltpu.SemaphoreType.DMA((NBUF,)),
        pltpu.SemaphoreType.DMA((NBUF,)),
        pltpu.SemaphoreType.DMA((NBUFI,)),
        pltpu.SemaphoreType.DMA((NBUFI,)),
    ],
)(_mp_body)


def kernel(x, edge_index):
    ei = edge_index.astype(jnp.int32)
    dst = ei[0]
    src = ei[1]
    zeros = jnp.zeros((ROWS_Z, DH), jnp.float32)
    return _mp_call(x, src, dst, zeros)
